# baseline (speedup 1.0000x reference)
import functools

import jax
import jax.numpy as jnp
from jax.experimental import pallas as pl
from jax.experimental.pallas import tpu as pltpu

N = 2
CH = 64
H = 64
W = 64
NH = 4
CHUNK = 144
C = 16
HB = 56
L = 2 * H * W
PAD = 16
K = (L + PAD) // CHUNK
LP = K * CHUNK
BH = N * NH


def _attn_body(xs_ref, ys_ref, fcoT_ref, out_ref, bs_ref):
    def chunk_step(k, _):
        kp = (k + K - 1) % K
        kn = (k + 1) % K
        q = xs_ref[0, pl.ds(k * CHUNK, CHUNK), :]
        xk = jnp.concatenate(
            [q,
             xs_ref[0, pl.ds(kp * CHUNK, CHUNK), :],
             xs_ref[0, pl.ds(kn * CHUNK, CHUNK), :]], axis=0)
        nrm = jnp.sqrt(jnp.sum(xk * xk, axis=1, keepdims=True))
        xm = xk / jnp.maximum(nrm, 5e-05)
        fco = jnp.concatenate(
            [fcoT_ref[0, k], fcoT_ref[0, kp], fcoT_ref[0, kn]],
            axis=1)
        raw = jax.lax.dot_general(
            q, xm, (((1,), (1,)), ((), ())),
            preferred_element_type=jnp.float32) + fco
        m = jnp.max(raw, axis=1, keepdims=True)
        e = jnp.exp(raw - m)
        s = jnp.sum(e, axis=1, keepdims=True)
        yk = jnp.concatenate(
            [ys_ref[0, pl.ds(k * CHUNK, CHUNK), :],
             ys_ref[0, pl.ds(kp * CHUNK, CHUNK), :],
             ys_ref[0, pl.ds(kn * CHUNK, CHUNK), :]], axis=0)
        ret = jax.lax.dot_general(
            e, yk, (((1,), (0,)), ((), ())),
            preferred_element_type=jnp.float32) / s
        out_ref[0, pl.ds(k * CHUNK, CHUNK), :] = ret
        bs_ref[0, pl.ds(k * CHUNK, CHUNK), :] = m + jnp.log(s)
        return 0

    jax.lax.fori_loop(0, K, chunk_step, 0)


def _chunked_attention(xs, ys, fcoT):
    return pl.pallas_call(
        _attn_body,
        grid=(BH,),
        in_specs=[
            pl.BlockSpec((1, LP, C), lambda b: (b, 0, 0)),
            pl.BlockSpec((1, LP, CH), lambda b: (b, 0, 0)),
            pl.BlockSpec((1, K, CHUNK, CHUNK), lambda b: (b, 0, 0, 0)),
        ],
        out_specs=[
            pl.BlockSpec((1, LP, CH), lambda b: (b, 0, 0)),
            pl.BlockSpec((1, LP, 1), lambda b: (b, 0, 0)),
        ],
        out_shape=[
            jax.ShapeDtypeStruct((BH, LP, CH), jnp.float32),
            jax.ShapeDtypeStruct((BH, LP, 1), jnp.float32),
        ],
    )(xs, ys, fcoT)


def _conv(x, w):
    return jax.lax.conv_general_dilated(
        x, w, (1, 1), 'SAME', dimension_numbers=('NCHW', 'OIHW', 'NCHW'))


def kernel(input1, input2, w_match, w_asm, w_asm_fc, fc_w1, fc_b1, fc_w2,
           fc_b2, rotations):
    n = input1.shape[0]
    hw = H * W

    x1 = _conv(input1, w_match).reshape(n, C, hw).transpose(0, 2, 1)
    x2 = _conv(input2, w_match).reshape(n, C, hw).transpose(0, 2, 1)
    x_embed = jnp.concatenate([x1, x2], axis=1)
    y1 = _conv(input1, w_asm).reshape(n, CH, hw).transpose(0, 2, 1)
    y2 = _conv(input2, w_asm).reshape(n, CH, hw).transpose(0, 2, 1)
    y_embed = jnp.concatenate([y1, y2], axis=1)
    f1 = _conv(input1, w_asm_fc).reshape(n, CH, hw).transpose(0, 2, 1)
    f2 = _conv(input2, w_asm_fc).reshape(n, CH, hw).transpose(0, 2, 1)
    fc_embed = jnp.concatenate([f1, f2], axis=1)

    hdn = jax.nn.relu(fc_embed @ fc_w1.T + fc_b1)
    fco = hdn @ fc_w2.T + fc_b2

    rotated = jnp.einsum('btf,fhi->bhti', x_embed, rotations)
    hash_codes = jnp.argmax(rotated, axis=-1)
    offsets = (jnp.arange(NH) * HB).reshape(1, -1, 1)
    hash_codes = (hash_codes + offsets).reshape(n, -1)
    indices = jnp.argsort(hash_codes, axis=-1)
    undo_sort = jnp.argsort(indices, axis=-1)
    mod_indices = indices % L

    def take(v):
        return jnp.take_along_axis(v, mod_indices[:, :, None], axis=1)

    def pad_chunks(v):
        d = v.shape[-1]
        vb = v.reshape(n, NH, L, d)
        vb = jnp.concatenate([vb, vb[:, :, -PAD:, :]], axis=2)
        return vb.reshape(BH, LP, d)

    xs = pad_chunks(take(x_embed))
    ys = pad_chunks(take(y_embed))
    fcoT = pad_chunks(take(fco)).reshape(BH, K, CHUNK, CHUNK).transpose(
        0, 1, 3, 2)

    ret_s, bs_s = _chunked_attention(xs, ys, fcoT)

    ret_s = ret_s.reshape(n, NH, LP, CH)[:, :, :L, :].reshape(n, NH * L, CH)
    bs_s = bs_s.reshape(n, NH, LP)[:, :, :L].reshape(n, NH * L)

    ret = jnp.take_along_axis(ret_s, undo_sort[:, :, None], axis=1)
    bs = jnp.take_along_axis(bs_s, undo_sort, axis=1)
    ret = ret.reshape(n, NH, L, CH)
    bs = bs.reshape(n, NH, L, 1)
    probs = jax.nn.softmax(bs, axis=1)
    ret = jnp.sum(ret * probs, axis=1)

    out1 = ret[:, :hw, :].transpose(0, 2, 1).reshape(n, CH, H, W) + input1
    out2 = ret[:, hw:, :].transpose(0, 2, 1).reshape(n, CH, H, W) + input2
    return (out1, out2)


# speedup vs baseline: 2.9875x; 2.9875x over previous
import functools

import jax
import jax.numpy as jnp
from jax.experimental import pallas as pl
from jax.experimental.pallas import tpu as pltpu

N = 2
CH = 64
H = 64
W = 64
NH = 4
CHUNK = 144
C = 16
HB = 56
L = 2 * H * W
PAD = 16
K = (L + PAD) // CHUNK
LP = K * CHUNK
BH = N * NH
PB = 128
NPB = L // PB
OC = 80


def _pos_body(oh_ref, pos_ref):
    b = pl.program_id(0)
    h = pl.program_id(1)
    ri = jax.lax.broadcasted_iota(jnp.int32, (PB, PB), 0)
    ci = jax.lax.broadcasted_iota(jnp.int32, (PB, PB), 1)
    t_le = (ri >= ci).astype(jnp.float32)
    vi = jax.lax.broadcasted_iota(jnp.int32, (HB, HB), 0)
    vj = jax.lax.broadcasted_iota(jnp.int32, (HB, HB), 1)
    t_lt = (vi < vj).astype(jnp.float32)

    oh_all = oh_ref[0, 0]
    tot = jnp.sum(oh_all, axis=0, keepdims=True)
    start = jax.lax.dot_general(
        tot, t_lt, (((1,), (0,)), ((), ())),
        precision=jax.lax.Precision.HIGHEST,
        preferred_element_type=jnp.float32)
    base = ((b * NH + h) * LP).astype(jnp.int32)

    def blk(t, run):
        ob = oh_ref[0, 0, pl.ds(t * PB, PB), :]
        rinc = jax.lax.dot_general(
            t_le, ob, (((1,), (0,)), ((), ())),
            precision=jax.lax.Precision.HIGHEST,
            preferred_element_type=jnp.float32)
        bias = run + start
        posf = jnp.sum(ob * (rinc - ob + bias), axis=1, keepdims=True)
        pos_ref[0, 0, pl.ds(t * PB, PB), :] = posf.astype(jnp.int32) + base
        return run + rinc[PB - 1:PB, :]

    jax.lax.fori_loop(0, NPB, blk, jnp.zeros((1, HB), jnp.float32))


def _positions(onehot):
    return pl.pallas_call(
        _pos_body,
        grid=(N, NH),
        in_specs=[pl.BlockSpec((1, 1, L, HB), lambda b, h: (b, h, 0, 0))],
        out_specs=pl.BlockSpec((1, 1, L, 1), lambda b, h: (b, h, 0, 0)),
        out_shape=jax.ShapeDtypeStruct((N, NH, L, 1), jnp.int32),
    )(onehot)


def _attn_body(xs_ref, ys_ref, fc_ref, out_ref):
    ident = (jax.lax.broadcasted_iota(jnp.int32, (CHUNK, CHUNK), 0) ==
             jax.lax.broadcasted_iota(jnp.int32, (CHUNK, CHUNK), 1)
             ).astype(jnp.float32)

    def load_chunk(ref, j):
        a = ref[0, pl.ds(j * CHUNK, CHUNK), :]
        tail = ref[0, pl.ds(L - PAD, PAD), :]
        fix = jnp.where(j == K - 1, tail, a[CHUNK - PAD:, :])
        return jnp.concatenate([a[:CHUNK - PAD, :], fix], axis=0)

    def chunk_step(k, _):
        kp = (k + K - 1) % K
        kn = (k + 1) % K
        q = load_chunk(xs_ref, k)
        xk = jnp.concatenate(
            [q, load_chunk(xs_ref, kp), load_chunk(xs_ref, kn)], axis=0)
        nrm = jnp.sqrt(jnp.sum(xk * xk, axis=1, keepdims=True))
        xm = xk / jnp.maximum(nrm, 5e-05)
        fc3 = jnp.concatenate(
            [load_chunk(fc_ref, k), load_chunk(fc_ref, kp),
             load_chunk(fc_ref, kn)], axis=0)
        raw_t = jax.lax.dot_general(
            xm, q, (((1,), (1,)), ((), ())),
            preferred_element_type=jnp.float32) + fc3
        m = jnp.max(raw_t, axis=0, keepdims=True)
        e = jnp.exp(raw_t - m)
        s = jnp.sum(e, axis=0, keepdims=True)
        en = e / s
        yk = jnp.concatenate(
            [load_chunk(ys_ref, k), load_chunk(ys_ref, kp),
             load_chunk(ys_ref, kn)], axis=0)
        ret = jax.lax.dot_general(
            en, yk, (((0,), (0,)), ((), ())),
            preferred_element_type=jnp.float32)
        bs_row = m + jnp.log(s)
        bs_col = jax.lax.dot_general(
            ident, bs_row, (((1,), (1,)), ((), ())),
            preferred_element_type=jnp.float32)
        payload = jnp.concatenate(
            [ret, jnp.broadcast_to(bs_col, (CHUNK, OC - CH))], axis=1)
        out_ref[0, pl.ds(k * CHUNK, CHUNK), :] = payload
        return 0

    jax.lax.fori_loop(0, K, chunk_step, 0)


def _chunked_attention(xs, ys, fcs):
    return pl.pallas_call(
        _attn_body,
        grid=(BH,),
        in_specs=[
            pl.BlockSpec((1, LP, C), lambda b: (b, 0, 0)),
            pl.BlockSpec((1, LP, CH), lambda b: (b, 0, 0)),
            pl.BlockSpec((1, LP, CHUNK), lambda b: (b, 0, 0)),
        ],
        out_specs=pl.BlockSpec((1, LP, OC), lambda b: (b, 0, 0)),
        out_shape=jax.ShapeDtypeStruct((BH, LP, OC), jnp.float32),
    )(xs, ys, fcs)


def _conv(x, w):
    return jax.lax.conv_general_dilated(
        x, w, (1, 1), 'SAME', dimension_numbers=('NCHW', 'OIHW', 'NCHW'))


def kernel(input1, input2, w_match, w_asm, w_asm_fc, fc_w1, fc_b1, fc_w2,
           fc_b2, rotations):
    n = input1.shape[0]
    hw = H * W

    x1 = _conv(input1, w_match).reshape(n, C, hw).transpose(0, 2, 1)
    x2 = _conv(input2, w_match).reshape(n, C, hw).transpose(0, 2, 1)
    x_embed = jnp.concatenate([x1, x2], axis=1)
    y1 = _conv(input1, w_asm).reshape(n, CH, hw).transpose(0, 2, 1)
    y2 = _conv(input2, w_asm).reshape(n, CH, hw).transpose(0, 2, 1)
    y_embed = jnp.concatenate([y1, y2], axis=1)
    f1 = _conv(input1, w_asm_fc).reshape(n, CH, hw).transpose(0, 2, 1)
    f2 = _conv(input2, w_asm_fc).reshape(n, CH, hw).transpose(0, 2, 1)
    fc_embed = jnp.concatenate([f1, f2], axis=1)

    hdn = jax.nn.relu(fc_embed @ fc_w1.T + fc_b1)
    fco = hdn @ fc_w2.T + fc_b2

    rotated = jnp.einsum('btf,fhi->bhti', x_embed, rotations)
    onehot = jax.nn.one_hot(jnp.argmax(rotated, axis=-1), HB, dtype=jnp.float32)

    pos = _positions(onehot)[..., 0]

    tok = jnp.arange(n * L, dtype=jnp.int32)
    tokb = jnp.broadcast_to(tok.reshape(n, 1, L), (n, NH, L))
    ind = jnp.zeros((BH * LP,), jnp.int32).at[pos.reshape(-1)].set(
        tokb.reshape(-1), mode='drop')
    xs = jnp.take(x_embed.reshape(n * L, C), ind, axis=0).reshape(BH, LP, C)
    ys = jnp.take(y_embed.reshape(n * L, CH), ind, axis=0).reshape(BH, LP, CH)
    fcs = jnp.take(fco.reshape(n * L, CHUNK), ind, axis=0).reshape(BH, LP, CHUNK)

    att = _chunked_attention(xs, ys, fcs)

    retf = jnp.take(att.reshape(BH * LP, OC), pos.reshape(-1), axis=0)
    retf = retf.reshape(n, NH, L, OC)
    ret = retf[..., :CH]
    bs = retf[..., CH:CH + 1]
    probs = jax.nn.softmax(bs, axis=1)
    ret = jnp.sum(ret * probs, axis=1)

    out1 = ret[:, :hw, :].transpose(0, 2, 1).reshape(n, CH, H, W) + input1
    out2 = ret[:, hw:, :].transpose(0, 2, 1).reshape(n, CH, H, W) + input2
    return (out1, out2)


# speedup vs baseline: 5.3886x; 1.8037x over previous
import functools

import jax
import jax.numpy as jnp
from jax import lax
from jax.experimental import pallas as pl
from jax.experimental.pallas import tpu as pltpu
from jax.experimental.pallas import tpu_sc as plsc

N = 2
CH = 64
H = 64
W = 64
NH = 4
CHUNK = 144
C = 16
HB = 56
L = 2 * H * W
PAD = 16
K = (L + PAD) // CHUNK
LP = K * CHUNK
BH = N * NH
PB = 128
NPB = L // PB
OC = 128


def _pos_body(oh_ref, pos_ref):
    b = pl.program_id(0)
    h = pl.program_id(1)
    ri = jax.lax.broadcasted_iota(jnp.int32, (PB, PB), 0)
    ci = jax.lax.broadcasted_iota(jnp.int32, (PB, PB), 1)
    t_le = (ri >= ci).astype(jnp.float32)
    vi = jax.lax.broadcasted_iota(jnp.int32, (HB, HB), 0)
    vj = jax.lax.broadcasted_iota(jnp.int32, (HB, HB), 1)
    t_lt = (vi < vj).astype(jnp.float32)

    oh_all = oh_ref[0, 0]
    tot = jnp.sum(oh_all, axis=0, keepdims=True)
    start = jax.lax.dot_general(
        tot, t_lt, (((1,), (0,)), ((), ())),
        precision=jax.lax.Precision.HIGHEST,
        preferred_element_type=jnp.float32)
    base = ((b * NH + h) * LP).astype(jnp.int32)

    def blk(t, run):
        ob = oh_ref[0, 0, pl.ds(t * PB, PB), :]
        rinc = jax.lax.dot_general(
            t_le, ob, (((1,), (0,)), ((), ())),
            precision=jax.lax.Precision.HIGHEST,
            preferred_element_type=jnp.float32)
        bias = run + start
        posf = jnp.sum(ob * (rinc - ob + bias), axis=1, keepdims=True)
        pos_ref[0, 0, pl.ds(t * PB, PB), :] = posf.astype(jnp.int32) + base
        return run + rinc[PB - 1:PB, :]

    jax.lax.fori_loop(0, NPB, blk, jnp.zeros((1, HB), jnp.float32))


def _positions(onehot):
    return pl.pallas_call(
        _pos_body,
        grid=(N, NH),
        in_specs=[pl.BlockSpec((1, 1, L, HB), lambda b, h: (b, h, 0, 0))],
        out_specs=pl.BlockSpec((1, 1, L, 1), lambda b, h: (b, h, 0, 0)),
        out_shape=jax.ShapeDtypeStruct((N, NH, L, 1), jnp.int32),
    )(onehot)


def _attn_body(srt_ref, out_ref):
    ident = (jax.lax.broadcasted_iota(jnp.int32, (CHUNK, CHUNK), 0) ==
             jax.lax.broadcasted_iota(jnp.int32, (CHUNK, CHUNK), 1)
             ).astype(jnp.float32)

    def load_chunk(j):
        a = srt_ref[0, pl.ds(j * CHUNK, CHUNK), :]
        tail = srt_ref[0, pl.ds(L - PAD, PAD), :]
        fix = jnp.where(j == K - 1, tail, a[CHUNK - PAD:, :])
        return jnp.concatenate([a[:CHUNK - PAD, :], fix], axis=0)

    def chunk_step(k, _):
        kp = (k + K - 1) % K
        kn = (k + 1) % K
        a_c = load_chunk(k)
        a3 = jnp.concatenate([a_c, load_chunk(kp), load_chunk(kn)], axis=0)
        q = a_c[:, :C]
        xk = a3[:, :C]
        yk = a3[:, C:C + CH]
        fc3 = a3[:, C + CH:C + CH + CHUNK]
        nrm = jnp.sqrt(jnp.sum(xk * xk, axis=1, keepdims=True))
        xm = xk / jnp.maximum(nrm, 5e-05)
        raw_t = jax.lax.dot_general(
            xm, q, (((1,), (1,)), ((), ())),
            preferred_element_type=jnp.float32) + fc3
        m = jnp.max(raw_t, axis=0, keepdims=True)
        e = jnp.exp(raw_t - m)
        s = jnp.sum(e, axis=0, keepdims=True)
        en = e / s
        ret = jax.lax.dot_general(
            en, yk, (((0,), (0,)), ((), ())),
            preferred_element_type=jnp.float32)
        bs_row = m + jnp.log(s)
        bs_col = jax.lax.dot_general(
            ident, bs_row, (((1,), (1,)), ((), ())),
            preferred_element_type=jnp.float32)
        payload = jnp.concatenate(
            [ret, jnp.broadcast_to(bs_col, (CHUNK, OC - CH))], axis=1)
        out_ref[0, pl.ds(k * CHUNK, CHUNK), :] = payload
        return 0

    jax.lax.fori_loop(0, K, chunk_step, 0)


def _chunked_attention(srt):
    return pl.pallas_call(
        _attn_body,
        grid=(BH,),
        in_specs=[pl.BlockSpec((1, LP, PAYW), lambda b: (b, 0, 0))],
        out_specs=pl.BlockSpec((1, LP, OC), lambda b: (b, 0, 0)),
        out_shape=jax.ShapeDtypeStruct((BH, LP, OC), jnp.float32),
    )(srt)


def _conv(x, w):
    return jax.lax.conv_general_dilated(
        x, w, (1, 1), 'SAME', dimension_numbers=('NCHW', 'OIHW', 'NCHW'))


NW = 32
SLICE = L // NW
IDXR = SLICE // 128
PAYW = 256

_SC_MESH = plsc.VectorSubcoreMesh(core_axis_name="c", subcore_axis_name="s")


def _sc_scatter_body(tok, pos, srt, idx_v, row_v, sem):
    wid = lax.axis_index("s") * 2 + lax.axis_index("c")
    for b in range(N):
        pltpu.sync_copy(tok.at[pl.ds(b * L + wid * SLICE, SLICE)], row_v)
        for h in range(NH):
            pltpu.sync_copy(pos.at[b, h, pl.ds(wid * IDXR, IDXR)], idx_v)
            handles = [
                pltpu.async_copy(row_v.at[pl.ds(j * 128, 128)],
                                 srt.at[idx_v.at[j]], sem)
                for j in range(IDXR)
            ]
            for hd in handles:
                hd.wait()


def _sc_scatter(tok, pos4):
    return pl.kernel(
        _sc_scatter_body,
        out_type=jax.ShapeDtypeStruct((BH * LP, PAYW), jnp.float32),
        mesh=_SC_MESH,
        scratch_types=[
            pltpu.VMEM((IDXR, 128), jnp.int32),
            pltpu.VMEM((SLICE, PAYW), jnp.float32),
            pltpu.SemaphoreType.DMA,
        ],
    )(tok, pos4)


def _sc_gather_body(att, pos, retf, idx_v, rows_v, sem):
    wid = lax.axis_index("s") * 2 + lax.axis_index("c")
    for b in range(N):
        for h in range(NH):
            pltpu.sync_copy(pos.at[b, h, pl.ds(wid * IDXR, IDXR)], idx_v)
            for j in range(IDXR):
                pltpu.async_copy(att.at[idx_v.at[j]], rows_v, sem).wait()
                dst = (b * NH + h) * L + wid * SLICE + j * 128
                pltpu.sync_copy(rows_v, retf.at[pl.ds(dst, 128)])


def _sc_gather(att, pos4):
    return pl.kernel(
        _sc_gather_body,
        out_type=jax.ShapeDtypeStruct((N * NH * L, OC), jnp.float32),
        mesh=_SC_MESH,
        scratch_types=[
            pltpu.VMEM((IDXR, 128), jnp.int32),
            pltpu.VMEM((128, OC), jnp.float32),
            pltpu.SemaphoreType.DMA,
        ],
    )(att, pos4)


CB = 512


def _comb_body(retf_ref, out_ref):
    lane = lax.broadcasted_iota(jnp.int32, (CB, OC), 1)
    msk = (lane == CH).astype(jnp.float32)
    rows = [retf_ref[0, h] for h in range(NH)]
    bs = [jnp.sum(r * msk, axis=1, keepdims=True) for r in rows]
    m = bs[0]
    for h in range(1, NH):
        m = jnp.maximum(m, bs[h])
    es = [jnp.exp(b - m) for b in bs]
    s = es[0]
    for h in range(1, NH):
        s = s + es[h]
    acc = jnp.zeros((CB, CH), jnp.float32)
    for h in range(NH):
        acc = acc + rows[h][:, :CH] * (es[h] / s)
    out_ref[0] = acc


def _combine(retf):
    return pl.pallas_call(
        _comb_body,
        grid=(N, L // CB),
        in_specs=[pl.BlockSpec((1, NH, CB, OC), lambda b, t: (b, 0, t, 0))],
        out_specs=pl.BlockSpec((1, CB, CH), lambda b, t: (b, t, 0)),
        out_shape=jax.ShapeDtypeStruct((N, L, CH), jnp.float32),
    )(retf)


def kernel(input1, input2, w_match, w_asm, w_asm_fc, fc_w1, fc_b1, fc_w2,
           fc_b2, rotations):
    n = input1.shape[0]
    hw = H * W

    x1 = _conv(input1, w_match).reshape(n, C, hw).transpose(0, 2, 1)
    x2 = _conv(input2, w_match).reshape(n, C, hw).transpose(0, 2, 1)
    x_embed = jnp.concatenate([x1, x2], axis=1)
    y1 = _conv(input1, w_asm).reshape(n, CH, hw).transpose(0, 2, 1)
    y2 = _conv(input2, w_asm).reshape(n, CH, hw).transpose(0, 2, 1)
    y_embed = jnp.concatenate([y1, y2], axis=1)
    f1 = _conv(input1, w_asm_fc).reshape(n, CH, hw).transpose(0, 2, 1)
    f2 = _conv(input2, w_asm_fc).reshape(n, CH, hw).transpose(0, 2, 1)
    fc_embed = jnp.concatenate([f1, f2], axis=1)

    hdn = jax.nn.relu(fc_embed @ fc_w1.T + fc_b1)
    fco = hdn @ fc_w2.T + fc_b2

    rotated = jnp.einsum('btf,fhi->bhti', x_embed, rotations)
    onehot = jax.nn.one_hot(jnp.argmax(rotated, axis=-1), HB, dtype=jnp.float32)

    pos4 = _positions(onehot).reshape(N, NH, L // 128, 128)

    tok = jnp.concatenate(
        [x_embed, y_embed, fco,
         jnp.zeros((n, L, PAYW - C - CH - CHUNK), jnp.float32)],
        axis=-1).reshape(n * L, PAYW)
    srt = _sc_scatter(tok, pos4)

    att = _chunked_attention(srt.reshape(BH, LP, PAYW))

    retf = _sc_gather(att.reshape(BH * LP, OC), pos4)
    ret = _combine(retf.reshape(n, NH, L, OC))

    out1 = ret[:, :hw, :].transpose(0, 2, 1).reshape(n, CH, H, W) + input1
    out2 = ret[:, hw:, :].transpose(0, 2, 1).reshape(n, CH, H, W) + input2
    return (out1, out2)


# speedup vs baseline: 5.9942x; 1.1124x over previous
import functools

import jax
import jax.numpy as jnp
from jax import lax
from jax.experimental import pallas as pl
from jax.experimental.pallas import tpu as pltpu
from jax.experimental.pallas import tpu_sc as plsc

N = 2
CH = 64
H = 64
W = 64
NH = 4
CHUNK = 144
C = 16
HB = 56
L = 2 * H * W
PAD = 16
K = (L + PAD) // CHUNK
LP = K * CHUNK
BH = N * NH
PB = 256
NPB = L // PB
OC = 128

FC0 = 0
X0 = FC0 + CHUNK
XN0 = X0 + C
Y0 = XN0 + C


def _pos_body(oh_ref, pos_ref):
    b = pl.program_id(0)
    h = pl.program_id(1)
    ri = jax.lax.broadcasted_iota(jnp.int32, (PB, PB), 0)
    ci = jax.lax.broadcasted_iota(jnp.int32, (PB, PB), 1)
    t_le = (ri >= ci).astype(jnp.float32)
    vi = jax.lax.broadcasted_iota(jnp.int32, (HB, HB), 0)
    vj = jax.lax.broadcasted_iota(jnp.int32, (HB, HB), 1)
    t_lt = (vi < vj).astype(jnp.float32)

    oh_all = oh_ref[0, 0]
    tot = jnp.sum(oh_all, axis=0, keepdims=True)
    start = jax.lax.dot_general(
        tot, t_lt, (((1,), (0,)), ((), ())),
        precision=jax.lax.Precision.HIGHEST,
        preferred_element_type=jnp.float32)
    base = ((b * NH + h) * LP).astype(jnp.int32)
    t_le_bf = t_le.astype(jnp.bfloat16)

    def blk(t, run):
        ob = oh_ref[0, 0, pl.ds(t * PB, PB), :]
        rinc = jax.lax.dot_general(
            t_le_bf, ob.astype(jnp.bfloat16), (((1,), (0,)), ((), ())),
            preferred_element_type=jnp.float32)
        bias = run + start
        posf = jnp.sum(ob * (rinc - ob + bias), axis=1, keepdims=True)
        pos_ref[0, 0, pl.ds(t * PB, PB), :] = posf.astype(jnp.int32) + base
        return run + rinc[PB - 1:PB, :]

    jax.lax.fori_loop(0, NPB, blk, jnp.zeros((1, HB), jnp.float32))


def _positions(onehot):
    return pl.pallas_call(
        _pos_body,
        grid=(N, NH),
        in_specs=[pl.BlockSpec((1, 1, L, HB), lambda b, h: (b, h, 0, 0))],
        out_specs=pl.BlockSpec((1, 1, L, 1), lambda b, h: (b, h, 0, 0)),
        out_shape=jax.ShapeDtypeStruct((N, NH, L, 1), jnp.int32),
    )(onehot)


def _attn_body(srt_ref, out_ref):
    ident = (jax.lax.broadcasted_iota(jnp.int32, (CHUNK, CHUNK), 0) ==
             jax.lax.broadcasted_iota(jnp.int32, (CHUNK, CHUNK), 1)
             ).astype(jnp.float32)

    def compute(a3, q, k):
        xm = a3[:, XN0:XN0 + C]
        yk = a3[:, Y0:Y0 + CH]
        fc3 = a3[:, FC0:FC0 + CHUNK]
        raw_t = jax.lax.dot_general(
            xm, q, (((1,), (1,)), ((), ())),
            preferred_element_type=jnp.float32) + fc3
        m = jnp.max(raw_t, axis=0, keepdims=True)
        e = jnp.exp(raw_t - m)
        s = jnp.sum(e, axis=0, keepdims=True)
        en = e / s
        ret = jax.lax.dot_general(
            en, yk, (((0,), (0,)), ((), ())),
            preferred_element_type=jnp.float32)
        bs_row = m + jnp.log(s)
        bs_col = jax.lax.dot_general(
            ident, bs_row, (((1,), (1,)), ((), ())),
            preferred_element_type=jnp.float32)
        payload = jnp.concatenate(
            [ret, jnp.broadcast_to(bs_col, (CHUNK, OC - CH))], axis=1)
        out_ref[0, pl.ds(k * CHUNK, CHUNK), :] = payload

    c56 = jnp.concatenate(
        [srt_ref[0, pl.ds((K - 1) * CHUNK, CHUNK - PAD), :],
         srt_ref[0, pl.ds(L - PAD, PAD), :]], axis=0)
    c0 = srt_ref[0, pl.ds(0, CHUNK), :]
    c1 = srt_ref[0, pl.ds(CHUNK, CHUNK), :]
    c55 = srt_ref[0, pl.ds((K - 2) * CHUNK, CHUNK), :]
    compute(jnp.concatenate([c56, c0, c1], axis=0), c0[:, X0:X0 + C], 0)
    compute(jnp.concatenate([c55, c56, c0], axis=0), c56[:, X0:X0 + C], K - 1)

    def chunk_step(k, _):
        a3 = srt_ref[0, pl.ds((k - 1) * CHUNK, 3 * CHUNK), :]
        q = a3[CHUNK:2 * CHUNK, X0:X0 + C]
        compute(a3, q, k)
        return 0

    jax.lax.fori_loop(1, K - 1, chunk_step, 0)


def _chunked_attention(srt):
    return pl.pallas_call(
        _attn_body,
        grid=(BH,),
        in_specs=[pl.BlockSpec((1, LP, PAYW), lambda b: (b, 0, 0))],
        out_specs=pl.BlockSpec((1, LP, OC), lambda b: (b, 0, 0)),
        out_shape=jax.ShapeDtypeStruct((BH, LP, OC), jnp.float32),
    )(srt)


def _conv(x, w):
    return jax.lax.conv_general_dilated(
        x, w, (1, 1), 'SAME', dimension_numbers=('NCHW', 'OIHW', 'NCHW'))


NW = 32
SLICE = L // NW
IDXR = SLICE // 128
PAYW = 256

_SC_MESH = plsc.VectorSubcoreMesh(core_axis_name="c", subcore_axis_name="s")


def _sc_scatter_body(tok, pos, srt, idx_v, row_v, sem):
    wid = lax.axis_index("s") * 2 + lax.axis_index("c")
    for b in range(N):
        pltpu.sync_copy(tok.at[pl.ds(b * L + wid * SLICE, SLICE)], row_v)
        for h in range(NH):
            pltpu.sync_copy(pos.at[b, h, pl.ds(wid * IDXR, IDXR)], idx_v)
            handles = [
                pltpu.async_copy(row_v.at[pl.ds(j * 128, 128)],
                                 srt.at[idx_v.at[j]], sem)
                for j in range(IDXR)
            ]
            for hd in handles:
                hd.wait()


def _sc_scatter(tok, pos4):
    return pl.kernel(
        _sc_scatter_body,
        out_type=jax.ShapeDtypeStruct((BH * LP, PAYW), jnp.float32),
        mesh=_SC_MESH,
        scratch_types=[
            pltpu.VMEM((IDXR, 128), jnp.int32),
            pltpu.VMEM((SLICE, PAYW), jnp.float32),
            pltpu.SemaphoreType.DMA,
        ],
    )(tok, pos4)


def _sc_gather_body(att, pos, retf, idx_v, rows_v, sem):
    wid = lax.axis_index("s") * 2 + lax.axis_index("c")
    for b in range(N):
        for h in range(NH):
            pltpu.sync_copy(pos.at[b, h, pl.ds(wid * IDXR, IDXR)], idx_v)
            for j in range(IDXR):
                pltpu.async_copy(att.at[idx_v.at[j]], rows_v, sem).wait()
                dst = (b * NH + h) * L + wid * SLICE + j * 128
                pltpu.sync_copy(rows_v, retf.at[pl.ds(dst, 128)])


def _sc_gather(att, pos4):
    return pl.kernel(
        _sc_gather_body,
        out_type=jax.ShapeDtypeStruct((N * NH * L, OC), jnp.float32),
        mesh=_SC_MESH,
        scratch_types=[
            pltpu.VMEM((IDXR, 128), jnp.int32),
            pltpu.VMEM((128, OC), jnp.float32),
            pltpu.SemaphoreType.DMA,
        ],
    )(att, pos4)


CB = 512


def _comb_body(retf_ref, out_ref):
    lane = lax.broadcasted_iota(jnp.int32, (CB, OC), 1)
    msk = (lane == CH).astype(jnp.float32)
    rows = [retf_ref[0, h] for h in range(NH)]
    bs = [jnp.sum(r * msk, axis=1, keepdims=True) for r in rows]
    m = bs[0]
    for h in range(1, NH):
        m = jnp.maximum(m, bs[h])
    es = [jnp.exp(b - m) for b in bs]
    s = es[0]
    for h in range(1, NH):
        s = s + es[h]
    acc = jnp.zeros((CB, CH), jnp.float32)
    for h in range(NH):
        acc = acc + rows[h][:, :CH] * (es[h] / s)
    out_ref[0] = acc


def _combine(retf):
    return pl.pallas_call(
        _comb_body,
        grid=(N, L // CB),
        in_specs=[pl.BlockSpec((1, NH, CB, OC), lambda b, t: (b, 0, t, 0))],
        out_specs=pl.BlockSpec((1, CB, CH), lambda b, t: (b, t, 0)),
        out_shape=jax.ShapeDtypeStruct((N, L, CH), jnp.float32),
    )(retf)


def kernel(input1, input2, w_match, w_asm, w_asm_fc, fc_w1, fc_b1, fc_w2,
           fc_b2, rotations):
    n = input1.shape[0]
    hw = H * W

    x1 = _conv(input1, w_match).reshape(n, C, hw).transpose(0, 2, 1)
    x2 = _conv(input2, w_match).reshape(n, C, hw).transpose(0, 2, 1)
    x_embed = jnp.concatenate([x1, x2], axis=1)
    y1 = _conv(input1, w_asm).reshape(n, CH, hw).transpose(0, 2, 1)
    y2 = _conv(input2, w_asm).reshape(n, CH, hw).transpose(0, 2, 1)
    y_embed = jnp.concatenate([y1, y2], axis=1)
    f1 = _conv(input1, w_asm_fc).reshape(n, CH, hw).transpose(0, 2, 1)
    f2 = _conv(input2, w_asm_fc).reshape(n, CH, hw).transpose(0, 2, 1)
    fc_embed = jnp.concatenate([f1, f2], axis=1)

    hdn = jax.nn.relu(fc_embed @ fc_w1.T + fc_b1)
    fco = hdn @ fc_w2.T + fc_b2

    rotated = jnp.einsum('btf,fhi->bhti', x_embed, rotations)
    onehot = jax.nn.one_hot(jnp.argmax(rotated, axis=-1), HB, dtype=jnp.float32)

    pos4 = _positions(onehot).reshape(N, NH, L // 128, 128)

    nrm = jnp.maximum(
        jnp.sqrt(jnp.sum(x_embed * x_embed, axis=-1, keepdims=True)), 5e-05)
    tok = jnp.concatenate(
        [fco, x_embed, x_embed / nrm, y_embed,
         jnp.zeros((n, L, PAYW - CHUNK - 2 * C - CH), jnp.float32)],
        axis=-1).reshape(n * L, PAYW)
    srt = _sc_scatter(tok, pos4)

    att = _chunked_attention(srt.reshape(BH, LP, PAYW))

    retf = _sc_gather(att.reshape(BH * LP, OC), pos4)
    ret = _combine(retf.reshape(n, NH, L, OC))

    out1 = ret[:, :hw, :].transpose(0, 2, 1).reshape(n, CH, H, W) + input1
    out2 = ret[:, hw:, :].transpose(0, 2, 1).reshape(n, CH, H, W) + input2
    return (out1, out2)


# speedup vs baseline: 6.0351x; 1.0068x over previous
import functools

import jax
import jax.numpy as jnp
from jax import lax
from jax.experimental import pallas as pl
from jax.experimental.pallas import tpu as pltpu
from jax.experimental.pallas import tpu_sc as plsc

N = 2
CH = 64
H = 64
W = 64
NH = 4
CHUNK = 144
C = 16
HB = 56
L = 2 * H * W
PAD = 16
K = (L + PAD) // CHUNK
LP = K * CHUNK
BH = N * NH
PB = 256
NPB = L // PB
OC = 128

FC0 = 0
X0 = FC0 + CHUNK
XN0 = X0 + C
Y0 = XN0 + C


def _pos_body(oh_ref, pos_ref):
    b = pl.program_id(0)
    h = pl.program_id(1)
    ri = jax.lax.broadcasted_iota(jnp.int32, (PB, PB), 0)
    ci = jax.lax.broadcasted_iota(jnp.int32, (PB, PB), 1)
    t_le = (ri >= ci).astype(jnp.float32)
    vi = jax.lax.broadcasted_iota(jnp.int32, (HB, HB), 0)
    vj = jax.lax.broadcasted_iota(jnp.int32, (HB, HB), 1)
    t_lt = (vi < vj).astype(jnp.float32)

    oh_all = oh_ref[0, 0]
    tot = jnp.sum(oh_all, axis=0, keepdims=True)
    start = jax.lax.dot_general(
        tot, t_lt, (((1,), (0,)), ((), ())),
        precision=jax.lax.Precision.HIGHEST,
        preferred_element_type=jnp.float32)
    base = ((b * NH + h) * LP).astype(jnp.int32)
    t_le_bf = t_le.astype(jnp.bfloat16)

    def blk(t, run):
        ob = oh_ref[0, 0, pl.ds(t * PB, PB), :]
        rinc = jax.lax.dot_general(
            t_le_bf, ob.astype(jnp.bfloat16), (((1,), (0,)), ((), ())),
            preferred_element_type=jnp.float32)
        bias = run + start
        posf = jnp.sum(ob * (rinc - ob + bias), axis=1, keepdims=True)
        pos_ref[0, 0, pl.ds(t * PB, PB), :] = posf.astype(jnp.int32) + base
        return run + rinc[PB - 1:PB, :]

    jax.lax.fori_loop(0, NPB, blk, jnp.zeros((1, HB), jnp.float32))


def _positions(onehot):
    return pl.pallas_call(
        _pos_body,
        grid=(N, NH),
        in_specs=[pl.BlockSpec((1, 1, L, HB), lambda b, h: (b, h, 0, 0))],
        out_specs=pl.BlockSpec((1, 1, L, 1), lambda b, h: (b, h, 0, 0)),
        out_shape=jax.ShapeDtypeStruct((N, NH, L, 1), jnp.int32),
    )(onehot)


def _attn_body(srt_ref, out_ref):
    ident = (jax.lax.broadcasted_iota(jnp.int32, (CHUNK, CHUNK), 0) ==
             jax.lax.broadcasted_iota(jnp.int32, (CHUNK, CHUNK), 1)
             ).astype(jnp.float32)

    def compute(a3, q, k):
        xm = a3[:, XN0:XN0 + C]
        yk = a3[:, Y0:Y0 + CH]
        fc3 = a3[:, FC0:FC0 + CHUNK]
        raw_t = jax.lax.dot_general(
            xm, q, (((1,), (1,)), ((), ())),
            preferred_element_type=jnp.float32) + fc3
        m = jnp.max(raw_t, axis=0, keepdims=True)
        e = jnp.exp(raw_t - m)
        s = jnp.sum(e, axis=0, keepdims=True)
        en = e / s
        ret = jax.lax.dot_general(
            en, yk, (((0,), (0,)), ((), ())),
            preferred_element_type=jnp.float32)
        bs_row = m + jnp.log(s)
        bs_col = jax.lax.dot_general(
            ident, bs_row, (((1,), (1,)), ((), ())),
            preferred_element_type=jnp.float32)
        payload = jnp.concatenate(
            [ret, jnp.broadcast_to(bs_col, (CHUNK, OC - CH))], axis=1)
        out_ref[0, pl.ds(k * CHUNK, CHUNK), :] = payload

    c56 = jnp.concatenate(
        [srt_ref[0, pl.ds((K - 1) * CHUNK, CHUNK - PAD), :],
         srt_ref[0, pl.ds(L - PAD, PAD), :]], axis=0)
    c0 = srt_ref[0, pl.ds(0, CHUNK), :]
    c1 = srt_ref[0, pl.ds(CHUNK, CHUNK), :]
    c54 = srt_ref[0, pl.ds((K - 3) * CHUNK, CHUNK), :]
    c55 = srt_ref[0, pl.ds((K - 2) * CHUNK, CHUNK), :]
    compute(jnp.concatenate([c56, c0, c1], axis=0), c0[:, X0:X0 + C], 0)
    compute(jnp.concatenate([c54, c55, c56], axis=0), c55[:, X0:X0 + C], K - 2)
    compute(jnp.concatenate([c55, c56, c0], axis=0), c56[:, X0:X0 + C], K - 1)

    def chunk_step(k, _):
        a3 = srt_ref[0, pl.ds((k - 1) * CHUNK, 3 * CHUNK), :]
        q = a3[CHUNK:2 * CHUNK, X0:X0 + C]
        compute(a3, q, k)
        return 0

    jax.lax.fori_loop(1, K - 2, chunk_step, 0)


def _chunked_attention(srt):
    return pl.pallas_call(
        _attn_body,
        grid=(BH,),
        in_specs=[pl.BlockSpec((1, LP, PAYW), lambda b: (b, 0, 0))],
        out_specs=pl.BlockSpec((1, LP, OC), lambda b: (b, 0, 0)),
        out_shape=jax.ShapeDtypeStruct((BH, LP, OC), jnp.float32),
    )(srt)


def _conv(x, w):
    return jax.lax.conv_general_dilated(
        x, w, (1, 1), 'SAME', dimension_numbers=('NCHW', 'OIHW', 'NCHW'))


NW = 32
SLICE = L // NW
IDXR = SLICE // 128
PAYW = 256

_SC_MESH = plsc.VectorSubcoreMesh(core_axis_name="c", subcore_axis_name="s")


def _sc_scatter_body(tok, pos, srt, idx_v, row_v, sem):
    wid = lax.axis_index("s") * 2 + lax.axis_index("c")
    for b in range(N):
        pltpu.sync_copy(tok.at[pl.ds(b * L + wid * SLICE, SLICE)], row_v)
        for h in range(NH):
            pltpu.sync_copy(pos.at[b, h, pl.ds(wid * IDXR, IDXR)], idx_v)
            handles = [
                pltpu.async_copy(row_v.at[pl.ds(j * 128, 128)],
                                 srt.at[idx_v.at[j]], sem)
                for j in range(IDXR)
            ]
            for hd in handles:
                hd.wait()


def _sc_scatter(tok, pos4):
    return pl.kernel(
        _sc_scatter_body,
        out_type=jax.ShapeDtypeStruct((BH * LP, PAYW), jnp.float32),
        mesh=_SC_MESH,
        scratch_types=[
            pltpu.VMEM((IDXR, 128), jnp.int32),
            pltpu.VMEM((SLICE, PAYW), jnp.float32),
            pltpu.SemaphoreType.DMA,
        ],
    )(tok, pos4)


def _sc_gather_body(att, pos, retf, idx_v, rows_v, sem):
    wid = lax.axis_index("s") * 2 + lax.axis_index("c")
    for b in range(N):
        for h in range(NH):
            pltpu.sync_copy(pos.at[b, h, pl.ds(wid * IDXR, IDXR)], idx_v)
            for j in range(IDXR):
                pltpu.async_copy(att.at[idx_v.at[j]], rows_v, sem).wait()
                dst = (b * NH + h) * L + wid * SLICE + j * 128
                pltpu.sync_copy(rows_v, retf.at[pl.ds(dst, 128)])


def _sc_gather(att, pos4):
    return pl.kernel(
        _sc_gather_body,
        out_type=jax.ShapeDtypeStruct((N * NH * L, OC), jnp.float32),
        mesh=_SC_MESH,
        scratch_types=[
            pltpu.VMEM((IDXR, 128), jnp.int32),
            pltpu.VMEM((128, OC), jnp.float32),
            pltpu.SemaphoreType.DMA,
        ],
    )(att, pos4)


CB = 512


def _comb_body(retf_ref, out_ref):
    lane = lax.broadcasted_iota(jnp.int32, (CB, OC), 1)
    msk = (lane == CH).astype(jnp.float32)
    rows = [retf_ref[0, h] for h in range(NH)]
    bs = [jnp.sum(r * msk, axis=1, keepdims=True) for r in rows]
    m = bs[0]
    for h in range(1, NH):
        m = jnp.maximum(m, bs[h])
    es = [jnp.exp(b - m) for b in bs]
    s = es[0]
    for h in range(1, NH):
        s = s + es[h]
    acc = jnp.zeros((CB, CH), jnp.float32)
    for h in range(NH):
        acc = acc + rows[h][:, :CH] * (es[h] / s)
    out_ref[0] = acc


def _combine(retf):
    return pl.pallas_call(
        _comb_body,
        grid=(N, L // CB),
        in_specs=[pl.BlockSpec((1, NH, CB, OC), lambda b, t: (b, 0, t, 0))],
        out_specs=pl.BlockSpec((1, CB, CH), lambda b, t: (b, t, 0)),
        out_shape=jax.ShapeDtypeStruct((N, L, CH), jnp.float32),
    )(retf)


def kernel(input1, input2, w_match, w_asm, w_asm_fc, fc_w1, fc_b1, fc_w2,
           fc_b2, rotations):
    n = input1.shape[0]
    hw = H * W

    x1 = _conv(input1, w_match).reshape(n, C, hw).transpose(0, 2, 1)
    x2 = _conv(input2, w_match).reshape(n, C, hw).transpose(0, 2, 1)
    x_embed = jnp.concatenate([x1, x2], axis=1)
    y1 = _conv(input1, w_asm).reshape(n, CH, hw).transpose(0, 2, 1)
    y2 = _conv(input2, w_asm).reshape(n, CH, hw).transpose(0, 2, 1)
    y_embed = jnp.concatenate([y1, y2], axis=1)
    f1 = _conv(input1, w_asm_fc).reshape(n, CH, hw).transpose(0, 2, 1)
    f2 = _conv(input2, w_asm_fc).reshape(n, CH, hw).transpose(0, 2, 1)
    fc_embed = jnp.concatenate([f1, f2], axis=1)

    hdn = jax.nn.relu(fc_embed @ fc_w1.T + fc_b1)
    fco = hdn @ fc_w2.T + fc_b2

    rotated = jnp.einsum('btf,fhi->bhti', x_embed, rotations)
    onehot = jax.nn.one_hot(jnp.argmax(rotated, axis=-1), HB, dtype=jnp.float32)

    pos4 = _positions(onehot).reshape(N, NH, L // 128, 128)

    nrm = jnp.maximum(
        jnp.sqrt(jnp.sum(x_embed * x_embed, axis=-1, keepdims=True)), 5e-05)
    tok = jnp.concatenate(
        [fco, x_embed, x_embed / nrm, y_embed,
         jnp.zeros((n, L, PAYW - CHUNK - 2 * C - CH), jnp.float32)],
        axis=-1).reshape(n * L, PAYW)
    srt = _sc_scatter(tok, pos4)

    att = _chunked_attention(srt.reshape(BH, LP, PAYW))

    retf = _sc_gather(att.reshape(BH * LP, OC), pos4)
    ret = _combine(retf.reshape(n, NH, L, OC))

    out1 = ret[:, :hw, :].transpose(0, 2, 1).reshape(n, CH, H, W) + input1
    out2 = ret[:, hw:, :].transpose(0, 2, 1).reshape(n, CH, H, W) + input2
    return (out1, out2)


# speedup vs baseline: 6.3963x; 1.0598x over previous
import functools

import jax
import jax.numpy as jnp
from jax import lax
from jax.experimental import pallas as pl
from jax.experimental.pallas import tpu as pltpu
from jax.experimental.pallas import tpu_sc as plsc

N = 2
CH = 64
H = 64
W = 64
NH = 4
CHUNK = 144
C = 16
HB = 56
L = 2 * H * W
PAD = 16
K = (L + PAD) // CHUNK
LP = K * CHUNK
BH = N * NH
PB = 256
NPB = L // PB
OC = 128

FC0 = 0
Y0 = 144
X0 = 208
XN0 = 224


def _pos_body(oh_ref, pos_ref):
    b = pl.program_id(0)
    h = pl.program_id(1)
    ri = jax.lax.broadcasted_iota(jnp.int32, (PB, PB), 0)
    ci = jax.lax.broadcasted_iota(jnp.int32, (PB, PB), 1)
    t_le = (ri >= ci).astype(jnp.float32)
    vi = jax.lax.broadcasted_iota(jnp.int32, (HB, HB), 0)
    vj = jax.lax.broadcasted_iota(jnp.int32, (HB, HB), 1)
    t_lt = (vi < vj).astype(jnp.float32)

    oh_all = oh_ref[0, 0]
    tot = jnp.sum(oh_all, axis=0, keepdims=True)
    start = jax.lax.dot_general(
        tot, t_lt, (((1,), (0,)), ((), ())),
        precision=jax.lax.Precision.HIGHEST,
        preferred_element_type=jnp.float32)
    base = ((b * NH + h) * LP).astype(jnp.int32)
    t_le_bf = t_le.astype(jnp.bfloat16)

    def blk(t, run):
        ob = oh_ref[0, 0, pl.ds(t * PB, PB), :]
        rinc = jax.lax.dot_general(
            t_le_bf, ob.astype(jnp.bfloat16), (((1,), (0,)), ((), ())),
            preferred_element_type=jnp.float32)
        bias = run + start
        posf = jnp.sum(ob * (rinc - ob + bias), axis=1, keepdims=True)
        pos_ref[0, 0, pl.ds(t * PB, PB), :] = posf.astype(jnp.int32) + base
        return run + rinc[PB - 1:PB, :]

    jax.lax.fori_loop(0, NPB, blk, jnp.zeros((1, HB), jnp.float32))


def _positions(onehot):
    return pl.pallas_call(
        _pos_body,
        grid=(N, NH),
        in_specs=[pl.BlockSpec((1, 1, L, HB), lambda b, h: (b, h, 0, 0))],
        out_specs=pl.BlockSpec((1, 1, L, 1), lambda b, h: (b, h, 0, 0)),
        out_shape=jax.ShapeDtypeStruct((N, NH, L, 1), jnp.int32),
    )(onehot)


def _attn_body(srt_ref, out_ref):
    ident = (jax.lax.broadcasted_iota(jnp.int32, (CHUNK, CHUNK), 0) ==
             jax.lax.broadcasted_iota(jnp.int32, (CHUNK, CHUNK), 1)
             ).astype(jnp.float32)

    def compute(a3, q, k):
        xm = a3[:, XN0:XN0 + C]
        yk = a3[:, Y0:Y0 + CH]
        fc3 = a3[:, FC0:FC0 + CHUNK]
        raw_t = jax.lax.dot_general(
            xm, q, (((1,), (1,)), ((), ())),
            preferred_element_type=jnp.float32) + fc3
        m = jnp.max(raw_t, axis=0, keepdims=True)
        e = jnp.exp(raw_t - m)
        s = jnp.sum(e, axis=0, keepdims=True)
        en = e * (1.0 / s)
        ret = jax.lax.dot_general(
            en, yk, (((0,), (0,)), ((), ())),
            preferred_element_type=jnp.float32)
        bs_row = m + jnp.log(s)
        bs_col = jax.lax.dot_general(
            ident, bs_row, (((1,), (1,)), ((), ())),
            preferred_element_type=jnp.float32)
        payload = jnp.concatenate(
            [ret, jnp.broadcast_to(bs_col, (CHUNK, OC - CH))], axis=1)
        out_ref[0, pl.ds(k * CHUNK, CHUNK), :] = payload

    c56 = jnp.concatenate(
        [srt_ref[0, pl.ds((K - 1) * CHUNK, CHUNK - PAD), :],
         srt_ref[0, pl.ds(L - PAD, PAD), :]], axis=0)
    c0 = srt_ref[0, pl.ds(0, CHUNK), :]
    c1 = srt_ref[0, pl.ds(CHUNK, CHUNK), :]
    c54 = srt_ref[0, pl.ds((K - 3) * CHUNK, CHUNK), :]
    c55 = srt_ref[0, pl.ds((K - 2) * CHUNK, CHUNK), :]
    compute(jnp.concatenate([c56, c0, c1], axis=0), c0[:, X0:X0 + C], 0)
    compute(jnp.concatenate([c54, c55, c56], axis=0), c55[:, X0:X0 + C], K - 2)
    compute(jnp.concatenate([c55, c56, c0], axis=0), c56[:, X0:X0 + C], K - 1)

    def chunk_pair(i, _):
        k = 1 + 2 * i
        a3 = srt_ref[0, pl.ds((k - 1) * CHUNK, 3 * CHUNK), :]
        compute(a3, a3[CHUNK:2 * CHUNK, X0:X0 + C], k)
        b3 = srt_ref[0, pl.ds(k * CHUNK, 3 * CHUNK), :]
        compute(b3, b3[CHUNK:2 * CHUNK, X0:X0 + C], k + 1)
        return 0

    jax.lax.fori_loop(0, (K - 3) // 2, chunk_pair, 0)


def _chunked_attention(srt):
    return pl.pallas_call(
        _attn_body,
        grid=(BH,),
        in_specs=[pl.BlockSpec((1, LP, PAYW), lambda b: (b, 0, 0))],
        out_specs=pl.BlockSpec((1, LP, OC), lambda b: (b, 0, 0)),
        out_shape=jax.ShapeDtypeStruct((BH, LP, OC), jnp.float32),
    )(srt)


def _conv(x, w):
    return jax.lax.conv_general_dilated(
        x, w, (1, 1), 'SAME', dimension_numbers=('NCHW', 'OIHW', 'NCHW'))


NW = 32
SLICE = L // NW
IDXR = SLICE // 128
PAYW = 256

_SC_MESH = plsc.VectorSubcoreMesh(core_axis_name="c", subcore_axis_name="s")


def _sc_scatter_body(tok, pos, srt, idx_v, row_v, sem):
    wid = lax.axis_index("s") * 2 + lax.axis_index("c")
    for b in range(N):
        pltpu.sync_copy(tok.at[pl.ds(b * L + wid * SLICE, SLICE)], row_v)
        for h in range(NH):
            pltpu.sync_copy(pos.at[b, h, pl.ds(wid * IDXR, IDXR)], idx_v)
            handles = [
                pltpu.async_copy(row_v.at[pl.ds(j * 128, 128)],
                                 srt.at[idx_v.at[j]], sem)
                for j in range(IDXR)
            ]
            for hd in handles:
                hd.wait()


def _sc_scatter(tok, pos4):
    return pl.kernel(
        _sc_scatter_body,
        out_type=jax.ShapeDtypeStruct((BH * LP, PAYW), jnp.float32),
        mesh=_SC_MESH,
        scratch_types=[
            pltpu.VMEM((IDXR, 128), jnp.int32),
            pltpu.VMEM((SLICE, PAYW), jnp.float32),
            pltpu.SemaphoreType.DMA,
        ],
    )(tok, pos4)


def _sc_gather_body(att, pos, retf, idx_v, rows_v, sem):
    wid = lax.axis_index("s") * 2 + lax.axis_index("c")
    for b in range(N):
        for h in range(NH):
            pltpu.sync_copy(pos.at[b, h, pl.ds(wid * IDXR, IDXR)], idx_v)
            for j in range(IDXR):
                pltpu.async_copy(att.at[idx_v.at[j]], rows_v, sem).wait()
                dst = (b * NH + h) * L + wid * SLICE + j * 128
                pltpu.sync_copy(rows_v, retf.at[pl.ds(dst, 128)])


def _sc_gather(att, pos4):
    return pl.kernel(
        _sc_gather_body,
        out_type=jax.ShapeDtypeStruct((N * NH * L, OC), jnp.float32),
        mesh=_SC_MESH,
        scratch_types=[
            pltpu.VMEM((IDXR, 128), jnp.int32),
            pltpu.VMEM((128, OC), jnp.float32),
            pltpu.SemaphoreType.DMA,
        ],
    )(att, pos4)


CB = 512


def _comb_body(retf_ref, out_ref):
    lane = lax.broadcasted_iota(jnp.int32, (CB, OC), 1)
    msk = (lane == CH).astype(jnp.float32)
    rows = [retf_ref[0, h] for h in range(NH)]
    bs = [jnp.sum(r * msk, axis=1, keepdims=True) for r in rows]
    m = bs[0]
    for h in range(1, NH):
        m = jnp.maximum(m, bs[h])
    es = [jnp.exp(b - m) for b in bs]
    s = es[0]
    for h in range(1, NH):
        s = s + es[h]
    acc = jnp.zeros((CB, CH), jnp.float32)
    for h in range(NH):
        acc = acc + rows[h][:, :CH] * (es[h] / s)
    out_ref[0] = acc


def _combine(retf):
    return pl.pallas_call(
        _comb_body,
        grid=(N, L // CB),
        in_specs=[pl.BlockSpec((1, NH, CB, OC), lambda b, t: (b, 0, t, 0))],
        out_specs=pl.BlockSpec((1, CB, CH), lambda b, t: (b, t, 0)),
        out_shape=jax.ShapeDtypeStruct((N, L, CH), jnp.float32),
    )(retf)


def kernel(input1, input2, w_match, w_asm, w_asm_fc, fc_w1, fc_b1, fc_w2,
           fc_b2, rotations):
    n = input1.shape[0]
    hw = H * W

    x1 = _conv(input1, w_match).reshape(n, C, hw).transpose(0, 2, 1)
    x2 = _conv(input2, w_match).reshape(n, C, hw).transpose(0, 2, 1)
    x_embed = jnp.concatenate([x1, x2], axis=1)
    y1 = _conv(input1, w_asm).reshape(n, CH, hw).transpose(0, 2, 1)
    y2 = _conv(input2, w_asm).reshape(n, CH, hw).transpose(0, 2, 1)
    y_embed = jnp.concatenate([y1, y2], axis=1)
    f1 = _conv(input1, w_asm_fc).reshape(n, CH, hw).transpose(0, 2, 1)
    f2 = _conv(input2, w_asm_fc).reshape(n, CH, hw).transpose(0, 2, 1)
    fc_embed = jnp.concatenate([f1, f2], axis=1)

    hdn = jax.nn.relu(fc_embed @ fc_w1.T + fc_b1)
    fco = hdn @ fc_w2.T + fc_b2

    rotated = jnp.einsum('btf,fhi->bhti', x_embed, rotations)
    onehot = jax.nn.one_hot(jnp.argmax(rotated, axis=-1), HB, dtype=jnp.float32)

    pos4 = _positions(onehot).reshape(N, NH, L // 128, 128)

    nrm = jnp.maximum(
        jnp.sqrt(jnp.sum(x_embed * x_embed, axis=-1, keepdims=True)), 5e-05)
    tok = jnp.concatenate(
        [fco, y_embed, x_embed, x_embed / nrm,
         jnp.zeros((n, L, PAYW - XN0 - C), jnp.float32)],
        axis=-1).reshape(n * L, PAYW)
    srt = _sc_scatter(tok, pos4)

    att = _chunked_attention(srt.reshape(BH, LP, PAYW))

    retf = _sc_gather(att.reshape(BH * LP, OC), pos4)
    ret = _combine(retf.reshape(n, NH, L, OC))

    out1 = ret[:, :hw, :].transpose(0, 2, 1).reshape(n, CH, H, W) + input1
    out2 = ret[:, hw:, :].transpose(0, 2, 1).reshape(n, CH, H, W) + input2
    return (out1, out2)
